# baseline (speedup 1.0000x reference)
import functools

import jax
import jax.numpy as jnp
from jax import lax
from jax.experimental import pallas as pl
from jax.experimental.pallas import tpu as pltpu
from jax.experimental.pallas import tpu_sc as plsc

D = 128
NV = 64
B = 1024
T = 200
P = B * T
NUM_CORES = 2
NUM_SUBCORES = 16
NW = NUM_CORES * NUM_SUBCORES
SLICES = 5
PS = P // SLICES
PW = PS // NW
CHUNK = 128
NCHUNK = PW // CHUNK
TC_BLK = 1024
NBLK_S = PS // TC_BLK
NBLK = P // TC_BLK


def _sc_gather(cid_hbm, table_hbm, ecat_hbm, idx_v, rows_v, gsem, wsem):
    wid = lax.axis_index("s") * NUM_CORES + lax.axis_index("c")
    base = wid * PW
    pltpu.sync_copy(cid_hbm.at[pl.ds(base, PW)], idx_v)

    def g_start(j, slot):
        pltpu.async_copy(
            table_hbm.at[idx_v.at[pl.ds(j * CHUNK, CHUNK)]],
            rows_v.at[slot], gsem)

    def g_wait(slot):
        pltpu.make_async_copy(
            table_hbm.at[idx_v.at[pl.ds(0, CHUNK)]],
            rows_v.at[slot], gsem).wait()

    def w_start(j, slot):
        pltpu.async_copy(
            rows_v.at[slot], ecat_hbm.at[pl.ds(base + j * CHUNK, CHUNK)],
            wsem)

    def w_wait(slot):
        pltpu.make_async_copy(
            rows_v.at[slot], ecat_hbm.at[pl.ds(base, CHUNK)], wsem).wait()

    g_start(0, 0)
    g_start(1, 1)

    def body(j, carry):
        slot = lax.rem(j, 4)

        @pl.when(j >= 2)
        def _():
            w_wait(lax.rem(j - 2, 4))

        @pl.when(j + 2 < NCHUNK)
        def _():
            g_start(j + 2, lax.rem(j + 2, 4))

        g_wait(slot)
        w_start(j, slot)
        return carry

    lax.fori_loop(0, NCHUNK, body, 0)
    w_wait(lax.rem(NCHUNK - 2, 4))
    w_wait(lax.rem(NCHUNK - 1, 4))


@functools.lru_cache(maxsize=None)
def _sc_gather_call():
    return pl.kernel(
        _sc_gather,
        out_type=jax.ShapeDtypeStruct((PS, D), jnp.float32),
        mesh=plsc.VectorSubcoreMesh(
            core_axis_name="c", subcore_axis_name="s",
            num_cores=NUM_CORES, num_subcores=NUM_SUBCORES,
        ),
        scratch_types=[
            pltpu.VMEM((PW,), jnp.int32),
            pltpu.VMEM((4, CHUNK, D), jnp.float32),
            pltpu.SemaphoreType.DMA,
            pltpu.SemaphoreType.DMA,
        ],
    )


def _tc_body(vidf_ref, cidf_ref, val_ref, ecat_ref, ones64_ref, m1_ref,
             m2_ref, ab2_ref, bc_ref, w2_ref, invd_ref, g_ref, be_ref,
             out_ref):
    vid = vidf_ref[:, :]
    cid = cidf_ref[:, :]
    val = val_ref[:, :]
    ones64 = ones64_ref[:, :]
    f32 = jnp.float32
    dot = lambda a, b: jnp.dot(a, b, preferred_element_type=f32)
    vidb = dot(vid, ones64)
    valb = dot(val, ones64)
    cidb = dot(cid, ones64)
    eq = vidb == lax.broadcasted_iota(jnp.int32, (TC_BLK, NV), 1).astype(f32)
    oh = jnp.where(eq, 1.0, 0.0)
    ohv = jnp.where(eq, valb, 0.0)
    ohc = jnp.where(jnp.logical_and(eq, cidb >= 0.0), 1.0, 0.0)
    h = jnp.maximum(dot(ohv, m1_ref[:, :]) + dot(oh, m2_ref[:, :]), 0.0)
    e_val = (dot(h, w2_ref[:, :]) + dot(oh, ab2_ref[:, :])
             + ecat_ref[:, :] * dot(ohc, bc_ref[:, :]))
    m = dot(e_val, invd_ref[:, 0:1])
    s2 = dot(e_val * e_val, invd_ref[:, 1:2])
    r = lax.rsqrt(s2 - m * m + 1e-5)
    rb = dot(r, g_ref[:, :])
    cb = dot(-m * r, g_ref[:, :])
    out_ref[:, :] = e_val * rb + (cb + be_ref[0:1, :])


def _small2d(shape):
    return pl.BlockSpec(shape, lambda i: (0,) * len(shape))


def _tc_specs(s, aliased):
    off = s * NBLK_S
    specs = [
        pl.BlockSpec((TC_BLK, 1), lambda i: (off + i, 0)),
        pl.BlockSpec((TC_BLK, 1), lambda i: (off + i, 0)),
        pl.BlockSpec((TC_BLK, 1), lambda i: (off + i, 0)),
        pl.BlockSpec((TC_BLK, D), lambda i: (i, 0)),
        _small2d((1, NV)),
        _small2d((NV, D)),
        _small2d((NV, D)),
        _small2d((NV, D)),
        _small2d((NV, D)),
        _small2d((D, D)),
        _small2d((D, 2)),
        _small2d((1, D)),
        _small2d((1, D)),
    ]
    if aliased:
        specs.append(pl.BlockSpec(memory_space=pl.ANY))
    return specs


def _tc_body_nop(*refs):
    _tc_body(*refs[:13], refs[-1])


def _tc_forward(e_cats, variate_ids, value_num, cat_ids, variate_type,
                numeric_means, numeric_stds, w1, b1, w2, b2, gamma, beta,
                interpret=False):
    f32 = jnp.float32
    isg = 1.0 / (numeric_stds + 1e-6)
    misg = numeric_means * isg
    an = (variate_type == 0).astype(f32)[:, None]
    ac = (variate_type == 1).astype(f32)[:, None]
    w1row = jnp.zeros((1, D), f32).at[0, :16].set(w1)
    b1row = jnp.zeros((1, D), f32).at[0, :16].set(b1)
    m1 = an * isg[:, None] * w1row
    m2 = an * (-misg[:, None] * w1row + b1row)
    ab2 = an * b2[None, :]
    bc = ac * jnp.ones((1, D), f32)
    w2p = jnp.zeros((D, D), f32).at[:16, :].set(w2)
    consts = (jnp.ones((1, NV), f32), m1, m2, ab2, bc, w2p,
              jnp.full((D, 2), 1.0 / D, f32),
              gamma.reshape(1, D), beta.reshape(1, D))
    vidf = variate_ids.astype(f32).reshape(P, 1)
    cidf = cat_ids.astype(f32).reshape(P, 1)
    valf = value_num.reshape(P, 1)

    out = None
    for s in range(SLICES):
        off = s * NBLK_S

        def omap(i, off=off):
            return (off + i, 0)

        aliased = out is not None
        call = pl.pallas_call(
            _tc_body_nop if aliased else _tc_body,
            grid=(NBLK_S,),
            in_specs=_tc_specs(s, aliased),
            out_specs=pl.BlockSpec((TC_BLK, D), omap),
            out_shape=jax.ShapeDtypeStruct((P, D), f32),
            input_output_aliases={13: 0} if aliased else {},
            interpret=interpret,
        )
        args = (vidf, cidf, valf, e_cats[s]) + consts
        if aliased:
            args = args + (out,)
        out = call(*args)
    return out


def kernel(variate_ids, value_num, cat_ids, variate_type, numeric_means,
           numeric_stds, w1, b1, w2, b2, cat_table, gamma, beta):
    gather_ids = jnp.maximum(cat_ids.reshape(P), 0)
    sc = _sc_gather_call()
    e_cats = [sc(lax.slice_in_dim(gather_ids, s * PS, (s + 1) * PS), cat_table)
              for s in range(SLICES)]

    out = _tc_forward(e_cats, variate_ids, value_num, cat_ids, variate_type,
                      numeric_means, numeric_stds, w1, b1, w2, b2, gamma,
                      beta)
    return out.reshape(B, T, D)


# speedup vs baseline: 1.1246x; 1.1246x over previous
import functools

import jax
import jax.numpy as jnp
from jax import lax
from jax.experimental import pallas as pl
from jax.experimental.pallas import tpu as pltpu
from jax.experimental.pallas import tpu_sc as plsc

D = 128
NV = 64
B = 1024
T = 200
P = B * T
NUM_CORES = 2
NUM_SUBCORES = 16
NW = NUM_CORES * NUM_SUBCORES
SLICES = 5
PS = P // SLICES
PW = PS // NW
CHUNK = 128
NCHUNK = PW // CHUNK
TC_BLK = 2048
NBLK_S = PS // TC_BLK
NBLK = P // TC_BLK


def _sc_gather(cid_hbm, table_hbm, ecat_hbm, idx_v, rows_v, gsem, wsem):
    wid = lax.axis_index("s") * NUM_CORES + lax.axis_index("c")
    base = wid * PW
    pltpu.sync_copy(cid_hbm.at[pl.ds(base, PW)], idx_v)

    def g_start(j, slot):
        pltpu.async_copy(
            table_hbm.at[idx_v.at[pl.ds(j * CHUNK, CHUNK)]],
            rows_v.at[slot], gsem)

    def g_wait(slot):
        pltpu.make_async_copy(
            table_hbm.at[idx_v.at[pl.ds(0, CHUNK)]],
            rows_v.at[slot], gsem).wait()

    def w_start(j, slot):
        pltpu.async_copy(
            rows_v.at[slot], ecat_hbm.at[pl.ds(base + j * CHUNK, CHUNK)],
            wsem)

    def w_wait(slot):
        pltpu.make_async_copy(
            rows_v.at[slot], ecat_hbm.at[pl.ds(base, CHUNK)], wsem).wait()

    g_start(0, 0)
    g_start(1, 1)

    def body(j, carry):
        slot = lax.rem(j, 4)

        @pl.when(j >= 2)
        def _():
            w_wait(lax.rem(j - 2, 4))

        @pl.when(j + 2 < NCHUNK)
        def _():
            g_start(j + 2, lax.rem(j + 2, 4))

        g_wait(slot)
        w_start(j, slot)
        return carry

    lax.fori_loop(0, NCHUNK, body, 0)
    w_wait(lax.rem(NCHUNK - 2, 4))
    w_wait(lax.rem(NCHUNK - 1, 4))


@functools.lru_cache(maxsize=None)
def _sc_gather_call():
    return pl.kernel(
        _sc_gather,
        out_type=jax.ShapeDtypeStruct((PS, D), jnp.float32),
        mesh=plsc.VectorSubcoreMesh(
            core_axis_name="c", subcore_axis_name="s",
            num_cores=NUM_CORES, num_subcores=NUM_SUBCORES,
        ),
        scratch_types=[
            pltpu.VMEM((PW,), jnp.int32),
            pltpu.VMEM((4, CHUNK, D), jnp.float32),
            pltpu.SemaphoreType.DMA,
            pltpu.SemaphoreType.DMA,
        ],
    )


def _tc_body(vidf_ref, cidf_ref, val_ref, ecat_ref, ones64_ref, m1_ref,
             m2_ref, ab2_ref, bc_ref, w2_ref, invd_ref, g_ref, be_ref,
             out_ref):
    vid = vidf_ref[:, :]
    cid = cidf_ref[:, :]
    val = val_ref[:, :]
    ones64 = ones64_ref[:, :]
    f32 = jnp.float32
    dot = lambda a, b: jnp.dot(a, b, preferred_element_type=f32)
    vidb = dot(vid, ones64)
    valb = dot(val, ones64)
    cidb = dot(cid, ones64)
    eq = vidb == lax.broadcasted_iota(jnp.int32, (TC_BLK, NV), 1).astype(f32)
    oh = jnp.where(eq, 1.0, 0.0)
    ohv = jnp.where(eq, valb, 0.0)
    ohc = jnp.where(jnp.logical_and(eq, cidb >= 0.0), 1.0, 0.0)
    h = jnp.maximum(dot(ohv, m1_ref[:, :]) + dot(oh, m2_ref[:, :]), 0.0)
    e_val = (dot(h, w2_ref[:, :]) + dot(oh, ab2_ref[:, :])
             + ecat_ref[:, :] * dot(ohc, bc_ref[:, :]))
    m = dot(e_val, invd_ref[:, 0:1])
    s2 = dot(e_val * e_val, invd_ref[:, 1:2])
    r = lax.rsqrt(s2 - m * m + 1e-5)
    rb = dot(r, g_ref[:, :])
    cb = dot(-m * r, g_ref[:, :])
    out_ref[:, :] = e_val * rb + (cb + be_ref[0:1, :])


def _small2d(shape):
    return pl.BlockSpec(shape, lambda i: (0,) * len(shape))


def _tc_specs(s, aliased):
    off = s * NBLK_S
    specs = [
        pl.BlockSpec((TC_BLK, 1), lambda i: (off + i, 0)),
        pl.BlockSpec((TC_BLK, 1), lambda i: (off + i, 0)),
        pl.BlockSpec((TC_BLK, 1), lambda i: (off + i, 0)),
        pl.BlockSpec((TC_BLK, D), lambda i: (i, 0)),
        _small2d((1, NV)),
        _small2d((NV, D)),
        _small2d((NV, D)),
        _small2d((NV, D)),
        _small2d((NV, D)),
        _small2d((D, D)),
        _small2d((D, 2)),
        _small2d((1, D)),
        _small2d((1, D)),
    ]
    if aliased:
        specs.append(pl.BlockSpec(memory_space=pl.ANY))
    return specs


def _tc_body_nop(*refs):
    _tc_body(*refs[:13], refs[-1])


def _tc_forward(e_cats, variate_ids, value_num, cat_ids, variate_type,
                numeric_means, numeric_stds, w1, b1, w2, b2, gamma, beta,
                interpret=False):
    f32 = jnp.float32
    isg = 1.0 / (numeric_stds + 1e-6)
    misg = numeric_means * isg
    an = (variate_type == 0).astype(f32)[:, None]
    ac = (variate_type == 1).astype(f32)[:, None]
    w1row = jnp.zeros((1, D), f32).at[0, :16].set(w1)
    b1row = jnp.zeros((1, D), f32).at[0, :16].set(b1)
    m1 = an * isg[:, None] * w1row
    m2 = an * (-misg[:, None] * w1row + b1row)
    ab2 = an * b2[None, :]
    bc = ac * jnp.ones((1, D), f32)
    w2p = jnp.zeros((D, D), f32).at[:16, :].set(w2)
    consts = (jnp.ones((1, NV), f32), m1, m2, ab2, bc, w2p,
              jnp.full((D, 2), 1.0 / D, f32),
              gamma.reshape(1, D), beta.reshape(1, D))
    vidf = variate_ids.astype(f32).reshape(P, 1)
    cidf = cat_ids.astype(f32).reshape(P, 1)
    valf = value_num.reshape(P, 1)

    out = None
    for s in range(SLICES):
        off = s * NBLK_S

        def omap(i, off=off):
            return (off + i, 0)

        aliased = out is not None
        call = pl.pallas_call(
            _tc_body_nop if aliased else _tc_body,
            grid=(NBLK_S,),
            in_specs=_tc_specs(s, aliased),
            out_specs=pl.BlockSpec((TC_BLK, D), omap),
            out_shape=jax.ShapeDtypeStruct((P, D), f32),
            input_output_aliases={13: 0} if aliased else {},
            interpret=interpret,
        )
        args = (vidf, cidf, valf, e_cats[s]) + consts
        if aliased:
            args = args + (out,)
        out = call(*args)
    return out


def kernel(variate_ids, value_num, cat_ids, variate_type, numeric_means,
           numeric_stds, w1, b1, w2, b2, cat_table, gamma, beta):
    gather_ids = jnp.maximum(cat_ids.reshape(P), 0)
    sc = _sc_gather_call()
    e_cats = [sc(lax.slice_in_dim(gather_ids, s * PS, (s + 1) * PS), cat_table)
              for s in range(SLICES)]

    out = _tc_forward(e_cats, variate_ids, value_num, cat_ids, variate_type,
                      numeric_means, numeric_stds, w1, b1, w2, b2, gamma,
                      beta)
    return out.reshape(B, T, D)


# speedup vs baseline: 1.1824x; 1.0514x over previous
import functools

import jax
import jax.numpy as jnp
from jax import lax
from jax.experimental import pallas as pl
from jax.experimental.pallas import tpu as pltpu
from jax.experimental.pallas import tpu_sc as plsc

D = 128
NV = 64
B = 1024
T = 200
P = B * T
NUM_CORES = 2
NUM_SUBCORES = 16
NW = NUM_CORES * NUM_SUBCORES
SLICES = 5
PS = P // SLICES
PW = PS // NW
CHUNK = 128
NCHUNK = PW // CHUNK
TC_BLK = 4096
NBLK_S = PS // TC_BLK
NBLK = P // TC_BLK


def _sc_gather(cid_hbm, table_hbm, ecat_hbm, idx_v, rows_v, gsem, wsem):
    wid = lax.axis_index("s") * NUM_CORES + lax.axis_index("c")
    base = wid * PW
    pltpu.sync_copy(cid_hbm.at[pl.ds(base, PW)], idx_v)

    def g_start(j, slot):
        pltpu.async_copy(
            table_hbm.at[idx_v.at[pl.ds(j * CHUNK, CHUNK)]],
            rows_v.at[slot], gsem)

    def g_wait(slot):
        pltpu.make_async_copy(
            table_hbm.at[idx_v.at[pl.ds(0, CHUNK)]],
            rows_v.at[slot], gsem).wait()

    def w_start(j, slot):
        pltpu.async_copy(
            rows_v.at[slot], ecat_hbm.at[pl.ds(base + j * CHUNK, CHUNK)],
            wsem)

    def w_wait(slot):
        pltpu.make_async_copy(
            rows_v.at[slot], ecat_hbm.at[pl.ds(base, CHUNK)], wsem).wait()

    g_start(0, 0)
    g_start(1, 1)

    def body(j, carry):
        slot = lax.rem(j, 4)

        @pl.when(j >= 2)
        def _():
            w_wait(lax.rem(j - 2, 4))

        @pl.when(j + 2 < NCHUNK)
        def _():
            g_start(j + 2, lax.rem(j + 2, 4))

        g_wait(slot)
        w_start(j, slot)
        return carry

    lax.fori_loop(0, NCHUNK, body, 0)
    w_wait(lax.rem(NCHUNK - 2, 4))
    w_wait(lax.rem(NCHUNK - 1, 4))


@functools.lru_cache(maxsize=None)
def _sc_gather_call():
    return pl.kernel(
        _sc_gather,
        out_type=jax.ShapeDtypeStruct((PS, D), jnp.float32),
        mesh=plsc.VectorSubcoreMesh(
            core_axis_name="c", subcore_axis_name="s",
            num_cores=NUM_CORES, num_subcores=NUM_SUBCORES,
        ),
        scratch_types=[
            pltpu.VMEM((PW,), jnp.int32),
            pltpu.VMEM((4, CHUNK, D), jnp.float32),
            pltpu.SemaphoreType.DMA,
            pltpu.SemaphoreType.DMA,
        ],
    )


def _tc_body(vidf_ref, cidf_ref, val_ref, ecat_ref, ones64_ref, m1_ref,
             m2_ref, ab2_ref, bc_ref, w2_ref, invd_ref, g_ref, be_ref,
             out_ref):
    vid = vidf_ref[:, :]
    cid = cidf_ref[:, :]
    val = val_ref[:, :]
    ones64 = ones64_ref[:, :]
    f32 = jnp.float32
    dot = lambda a, b: jnp.dot(a, b, preferred_element_type=f32)
    vidb = dot(vid, ones64)
    valb = dot(val, ones64)
    cidb = dot(cid, ones64)
    eq = vidb == lax.broadcasted_iota(jnp.int32, (TC_BLK, NV), 1).astype(f32)
    oh = jnp.where(eq, 1.0, 0.0)
    ohv = jnp.where(eq, valb, 0.0)
    ohc = jnp.where(jnp.logical_and(eq, cidb >= 0.0), 1.0, 0.0)
    h = jnp.maximum(dot(ohv, m1_ref[:, :]) + dot(oh, m2_ref[:, :]), 0.0)
    e_val = (dot(h, w2_ref[:, :]) + dot(oh, ab2_ref[:, :])
             + ecat_ref[:, :] * dot(ohc, bc_ref[:, :]))
    m = dot(e_val, invd_ref[:, 0:1])
    s2 = dot(e_val * e_val, invd_ref[:, 1:2])
    r = lax.rsqrt(s2 - m * m + 1e-5)
    rb = dot(r, g_ref[:, :])
    cb = dot(-m * r, g_ref[:, :])
    out_ref[:, :] = e_val * rb + (cb + be_ref[0:1, :])


def _small2d(shape):
    return pl.BlockSpec(shape, lambda i: (0,) * len(shape))


def _tc_specs(s, aliased):
    off = s * NBLK_S
    specs = [
        pl.BlockSpec((TC_BLK, 1), lambda i: (off + i, 0)),
        pl.BlockSpec((TC_BLK, 1), lambda i: (off + i, 0)),
        pl.BlockSpec((TC_BLK, 1), lambda i: (off + i, 0)),
        pl.BlockSpec((TC_BLK, D), lambda i: (i, 0)),
        _small2d((1, NV)),
        _small2d((NV, D)),
        _small2d((NV, D)),
        _small2d((NV, D)),
        _small2d((NV, D)),
        _small2d((D, D)),
        _small2d((D, 2)),
        _small2d((1, D)),
        _small2d((1, D)),
    ]
    if aliased:
        specs.append(pl.BlockSpec(memory_space=pl.ANY))
    return specs


def _tc_body_nop(*refs):
    _tc_body(*refs[:13], refs[-1])


def _tc_forward(e_cats, variate_ids, value_num, cat_ids, variate_type,
                numeric_means, numeric_stds, w1, b1, w2, b2, gamma, beta,
                interpret=False):
    f32 = jnp.float32
    isg = 1.0 / (numeric_stds + 1e-6)
    misg = numeric_means * isg
    an = (variate_type == 0).astype(f32)[:, None]
    ac = (variate_type == 1).astype(f32)[:, None]
    w1row = jnp.zeros((1, D), f32).at[0, :16].set(w1)
    b1row = jnp.zeros((1, D), f32).at[0, :16].set(b1)
    m1 = an * isg[:, None] * w1row
    m2 = an * (-misg[:, None] * w1row + b1row)
    ab2 = an * b2[None, :]
    bc = ac * jnp.ones((1, D), f32)
    w2p = jnp.zeros((D, D), f32).at[:16, :].set(w2)
    consts = (jnp.ones((1, NV), f32), m1, m2, ab2, bc, w2p,
              jnp.full((D, 2), 1.0 / D, f32),
              gamma.reshape(1, D), beta.reshape(1, D))
    vidf = variate_ids.astype(f32).reshape(P, 1)
    cidf = cat_ids.astype(f32).reshape(P, 1)
    valf = value_num.reshape(P, 1)

    out = None
    for s in range(SLICES):
        off = s * NBLK_S

        def omap(i, off=off):
            return (off + i, 0)

        aliased = out is not None
        call = pl.pallas_call(
            _tc_body_nop if aliased else _tc_body,
            grid=(NBLK_S,),
            in_specs=_tc_specs(s, aliased),
            out_specs=pl.BlockSpec((TC_BLK, D), omap),
            out_shape=jax.ShapeDtypeStruct((P, D), f32),
            input_output_aliases={13: 0} if aliased else {},
            interpret=interpret,
        )
        args = (vidf, cidf, valf, e_cats[s]) + consts
        if aliased:
            args = args + (out,)
        out = call(*args)
    return out


def kernel(variate_ids, value_num, cat_ids, variate_type, numeric_means,
           numeric_stds, w1, b1, w2, b2, cat_table, gamma, beta):
    gather_ids = jnp.maximum(cat_ids.reshape(P), 0)
    sc = _sc_gather_call()
    e_cats = [sc(lax.slice_in_dim(gather_ids, s * PS, (s + 1) * PS), cat_table)
              for s in range(SLICES)]

    out = _tc_forward(e_cats, variate_ids, value_num, cat_ids, variate_type,
                      numeric_means, numeric_stds, w1, b1, w2, b2, gamma,
                      beta)
    return out.reshape(B, T, D)


# speedup vs baseline: 1.1868x; 1.0038x over previous
import functools

import jax
import jax.numpy as jnp
from jax import lax
from jax.experimental import pallas as pl
from jax.experimental.pallas import tpu as pltpu
from jax.experimental.pallas import tpu_sc as plsc

D = 128
NV = 64
B = 1024
T = 200
P = B * T
NUM_CORES = 2
NUM_SUBCORES = 16
NW = NUM_CORES * NUM_SUBCORES
SLICES = 5
PS = P // SLICES
PW = PS // NW
CHUNK = 128
NCHUNK = PW // CHUNK
TC_BLK = 5120
NBLK_S = PS // TC_BLK
NBLK = P // TC_BLK


def _sc_gather(cid_hbm, table_hbm, ecat_hbm, idx_v, rows_v, gsem, wsem):
    wid = lax.axis_index("s") * NUM_CORES + lax.axis_index("c")
    base = wid * PW
    pltpu.sync_copy(cid_hbm.at[pl.ds(base, PW)], idx_v)

    def g_start(j, slot):
        pltpu.async_copy(
            table_hbm.at[idx_v.at[pl.ds(j * CHUNK, CHUNK)]],
            rows_v.at[slot], gsem)

    def g_wait(slot):
        pltpu.make_async_copy(
            table_hbm.at[idx_v.at[pl.ds(0, CHUNK)]],
            rows_v.at[slot], gsem).wait()

    def w_start(j, slot):
        pltpu.async_copy(
            rows_v.at[slot], ecat_hbm.at[pl.ds(base + j * CHUNK, CHUNK)],
            wsem)

    def w_wait(slot):
        pltpu.make_async_copy(
            rows_v.at[slot], ecat_hbm.at[pl.ds(base, CHUNK)], wsem).wait()

    g_start(0, 0)
    g_start(1, 1)

    def body(j, carry):
        slot = lax.rem(j, 4)

        @pl.when(j >= 2)
        def _():
            w_wait(lax.rem(j - 2, 4))

        @pl.when(j + 2 < NCHUNK)
        def _():
            g_start(j + 2, lax.rem(j + 2, 4))

        g_wait(slot)
        w_start(j, slot)
        return carry

    lax.fori_loop(0, NCHUNK, body, 0)
    w_wait(lax.rem(NCHUNK - 2, 4))
    w_wait(lax.rem(NCHUNK - 1, 4))


@functools.lru_cache(maxsize=None)
def _sc_gather_call():
    return pl.kernel(
        _sc_gather,
        out_type=jax.ShapeDtypeStruct((PS, D), jnp.float32),
        mesh=plsc.VectorSubcoreMesh(
            core_axis_name="c", subcore_axis_name="s",
            num_cores=NUM_CORES, num_subcores=NUM_SUBCORES,
        ),
        scratch_types=[
            pltpu.VMEM((PW,), jnp.int32),
            pltpu.VMEM((4, CHUNK, D), jnp.float32),
            pltpu.SemaphoreType.DMA,
            pltpu.SemaphoreType.DMA,
        ],
    )


def _tc_body(vidf_ref, cidf_ref, val_ref, ecat_ref, ones64_ref, m1_ref,
             m2_ref, ab2_ref, bc_ref, w2_ref, invd_ref, g_ref, be_ref,
             out_ref):
    vid = vidf_ref[:, :]
    cid = cidf_ref[:, :]
    val = val_ref[:, :]
    ones64 = ones64_ref[:, :]
    f32 = jnp.float32
    dot = lambda a, b: jnp.dot(a, b, preferred_element_type=f32)
    vidb = dot(vid, ones64)
    valb = dot(val, ones64)
    cidb = dot(cid, ones64)
    eq = vidb == lax.broadcasted_iota(jnp.int32, (TC_BLK, NV), 1).astype(f32)
    oh = jnp.where(eq, 1.0, 0.0)
    ohv = jnp.where(eq, valb, 0.0)
    ohc = jnp.where(jnp.logical_and(eq, cidb >= 0.0), 1.0, 0.0)
    h = jnp.maximum(dot(ohv, m1_ref[:, :]) + dot(oh, m2_ref[:, :]), 0.0)
    e_val = (dot(h, w2_ref[:, :]) + dot(oh, ab2_ref[:, :])
             + ecat_ref[:, :] * dot(ohc, bc_ref[:, :]))
    m = dot(e_val, invd_ref[:, 0:1])
    s2 = dot(e_val * e_val, invd_ref[:, 1:2])
    r = lax.rsqrt(s2 - m * m + 1e-5)
    rb = dot(r, g_ref[:, :])
    cb = dot(-m * r, g_ref[:, :])
    out_ref[:, :] = e_val * rb + (cb + be_ref[0:1, :])


def _small2d(shape):
    return pl.BlockSpec(shape, lambda i: (0,) * len(shape))


def _tc_specs(s, aliased):
    off = s * NBLK_S
    specs = [
        pl.BlockSpec((TC_BLK, 1), lambda i: (off + i, 0)),
        pl.BlockSpec((TC_BLK, 1), lambda i: (off + i, 0)),
        pl.BlockSpec((TC_BLK, 1), lambda i: (off + i, 0)),
        pl.BlockSpec((TC_BLK, D), lambda i: (i, 0)),
        _small2d((1, NV)),
        _small2d((NV, D)),
        _small2d((NV, D)),
        _small2d((NV, D)),
        _small2d((NV, D)),
        _small2d((D, D)),
        _small2d((D, 2)),
        _small2d((1, D)),
        _small2d((1, D)),
    ]
    if aliased:
        specs.append(pl.BlockSpec(memory_space=pl.ANY))
    return specs


def _tc_body_nop(*refs):
    _tc_body(*refs[:13], refs[-1])


def _tc_forward(e_cats, variate_ids, value_num, cat_ids, variate_type,
                numeric_means, numeric_stds, w1, b1, w2, b2, gamma, beta,
                interpret=False):
    f32 = jnp.float32
    isg = 1.0 / (numeric_stds + 1e-6)
    misg = numeric_means * isg
    an = (variate_type == 0).astype(f32)[:, None]
    ac = (variate_type == 1).astype(f32)[:, None]
    w1row = jnp.zeros((1, D), f32).at[0, :16].set(w1)
    b1row = jnp.zeros((1, D), f32).at[0, :16].set(b1)
    m1 = an * isg[:, None] * w1row
    m2 = an * (-misg[:, None] * w1row + b1row)
    ab2 = an * b2[None, :]
    bc = ac * jnp.ones((1, D), f32)
    w2p = jnp.zeros((D, D), f32).at[:16, :].set(w2)
    consts = (jnp.ones((1, NV), f32), m1, m2, ab2, bc, w2p,
              jnp.full((D, 2), 1.0 / D, f32),
              gamma.reshape(1, D), beta.reshape(1, D))
    vidf = variate_ids.astype(f32).reshape(P, 1)
    cidf = cat_ids.astype(f32).reshape(P, 1)
    valf = value_num.reshape(P, 1)

    out = None
    for s in range(SLICES):
        off = s * NBLK_S

        def omap(i, off=off):
            return (off + i, 0)

        aliased = out is not None
        call = pl.pallas_call(
            _tc_body_nop if aliased else _tc_body,
            grid=(NBLK_S,),
            in_specs=_tc_specs(s, aliased),
            out_specs=pl.BlockSpec((TC_BLK, D), omap),
            out_shape=jax.ShapeDtypeStruct((P, D), f32),
            input_output_aliases={13: 0} if aliased else {},
            interpret=interpret,
        )
        args = (vidf, cidf, valf, e_cats[s]) + consts
        if aliased:
            args = args + (out,)
        out = call(*args)
    return out


def kernel(variate_ids, value_num, cat_ids, variate_type, numeric_means,
           numeric_stds, w1, b1, w2, b2, cat_table, gamma, beta):
    gather_ids = jnp.maximum(cat_ids.reshape(P), 0)
    sc = _sc_gather_call()
    e_cats = [sc(lax.slice_in_dim(gather_ids, s * PS, (s + 1) * PS), cat_table)
              for s in range(SLICES)]

    out = _tc_forward(e_cats, variate_ids, value_num, cat_ids, variate_type,
                      numeric_means, numeric_stds, w1, b1, w2, b2, gamma,
                      beta)
    return out.reshape(B, T, D)


# speedup vs baseline: 1.1909x; 1.0035x over previous
import functools

import jax
import jax.numpy as jnp
from jax import lax
from jax.experimental import pallas as pl
from jax.experimental.pallas import tpu as pltpu
from jax.experimental.pallas import tpu_sc as plsc

D = 128
NV = 64
B = 1024
T = 200
P = B * T
NUM_CORES = 2
NUM_SUBCORES = 16
NW = NUM_CORES * NUM_SUBCORES
SLICES = 5
PS = P // SLICES
PW = PS // NW
CHUNK = 128
NSLOT = 6
NPRE = NSLOT - 2
NCHUNK = PW // CHUNK
TC_BLK = 5120
NBLK_S = PS // TC_BLK
NBLK = P // TC_BLK


def _sc_gather(cid_hbm, table_hbm, ecat_hbm, idx_v, rows_v, gsem, wsem):
    wid = lax.axis_index("s") * NUM_CORES + lax.axis_index("c")
    base = wid * PW
    pltpu.sync_copy(cid_hbm.at[pl.ds(base, PW)], idx_v)

    def g_start(j, slot):
        pltpu.async_copy(
            table_hbm.at[idx_v.at[pl.ds(j * CHUNK, CHUNK)]],
            rows_v.at[slot], gsem)

    def g_wait(slot):
        pltpu.make_async_copy(
            table_hbm.at[idx_v.at[pl.ds(0, CHUNK)]],
            rows_v.at[slot], gsem).wait()

    def w_start(j, slot):
        pltpu.async_copy(
            rows_v.at[slot], ecat_hbm.at[pl.ds(base + j * CHUNK, CHUNK)],
            wsem)

    def w_wait(slot):
        pltpu.make_async_copy(
            rows_v.at[slot], ecat_hbm.at[pl.ds(base, CHUNK)], wsem).wait()

    for k in range(min(NPRE, NCHUNK)):
        g_start(k, k % NSLOT)

    def body(j, carry):
        slot = lax.rem(j, NSLOT)

        @pl.when(j >= 2)
        def _():
            w_wait(lax.rem(j - 2, NSLOT))

        @pl.when(j + NPRE < NCHUNK)
        def _():
            g_start(j + NPRE, lax.rem(j + NPRE, NSLOT))

        g_wait(slot)
        w_start(j, slot)
        return carry

    lax.fori_loop(0, NCHUNK, body, 0)
    w_wait(lax.rem(NCHUNK - 2, NSLOT))
    w_wait(lax.rem(NCHUNK - 1, NSLOT))


@functools.lru_cache(maxsize=None)
def _sc_gather_call():
    return pl.kernel(
        _sc_gather,
        out_type=jax.ShapeDtypeStruct((PS, D), jnp.float32),
        mesh=plsc.VectorSubcoreMesh(
            core_axis_name="c", subcore_axis_name="s",
            num_cores=NUM_CORES, num_subcores=NUM_SUBCORES,
        ),
        scratch_types=[
            pltpu.VMEM((PW,), jnp.int32),
            pltpu.VMEM((NSLOT, CHUNK, D), jnp.float32),
            pltpu.SemaphoreType.DMA,
            pltpu.SemaphoreType.DMA,
        ],
    )


def _tc_body(vidf_ref, cidf_ref, val_ref, ecat_ref, ones64_ref, m1_ref,
             m2_ref, ab2_ref, bc_ref, w2_ref, invd_ref, g_ref, be_ref,
             out_ref):
    vid = vidf_ref[:, :]
    cid = cidf_ref[:, :]
    val = val_ref[:, :]
    ones64 = ones64_ref[:, :]
    f32 = jnp.float32
    dot = lambda a, b: jnp.dot(a, b, preferred_element_type=f32)
    vidb = dot(vid, ones64)
    valb = dot(val, ones64)
    cidb = dot(cid, ones64)
    eq = vidb == lax.broadcasted_iota(jnp.int32, (TC_BLK, NV), 1).astype(f32)
    oh = jnp.where(eq, 1.0, 0.0)
    ohv = jnp.where(eq, valb, 0.0)
    ohc = jnp.where(jnp.logical_and(eq, cidb >= 0.0), 1.0, 0.0)
    h = jnp.maximum(dot(ohv, m1_ref[:, :]) + dot(oh, m2_ref[:, :]), 0.0)
    e_val = (dot(h, w2_ref[:, :]) + dot(oh, ab2_ref[:, :])
             + ecat_ref[:, :] * dot(ohc, bc_ref[:, :]))
    m = dot(e_val, invd_ref[:, 0:1])
    s2 = dot(e_val * e_val, invd_ref[:, 1:2])
    r = lax.rsqrt(s2 - m * m + 1e-5)
    rb = dot(r, g_ref[:, :])
    cb = dot(-m * r, g_ref[:, :])
    out_ref[:, :] = e_val * rb + (cb + be_ref[0:1, :])


def _small2d(shape):
    return pl.BlockSpec(shape, lambda i: (0,) * len(shape))


def _tc_specs(s, aliased):
    off = s * NBLK_S
    specs = [
        pl.BlockSpec((TC_BLK, 1), lambda i: (off + i, 0)),
        pl.BlockSpec((TC_BLK, 1), lambda i: (off + i, 0)),
        pl.BlockSpec((TC_BLK, 1), lambda i: (off + i, 0)),
        pl.BlockSpec((TC_BLK, D), lambda i: (i, 0)),
        _small2d((1, NV)),
        _small2d((NV, D)),
        _small2d((NV, D)),
        _small2d((NV, D)),
        _small2d((NV, D)),
        _small2d((D, D)),
        _small2d((D, 2)),
        _small2d((1, D)),
        _small2d((1, D)),
    ]
    if aliased:
        specs.append(pl.BlockSpec(memory_space=pl.ANY))
    return specs


def _tc_body_nop(*refs):
    _tc_body(*refs[:13], refs[-1])


def _tc_forward(e_cats, variate_ids, value_num, cat_ids, variate_type,
                numeric_means, numeric_stds, w1, b1, w2, b2, gamma, beta,
                interpret=False):
    f32 = jnp.float32
    isg = 1.0 / (numeric_stds + 1e-6)
    misg = numeric_means * isg
    an = (variate_type == 0).astype(f32)[:, None]
    ac = (variate_type == 1).astype(f32)[:, None]
    w1row = jnp.zeros((1, D), f32).at[0, :16].set(w1)
    b1row = jnp.zeros((1, D), f32).at[0, :16].set(b1)
    m1 = an * isg[:, None] * w1row
    m2 = an * (-misg[:, None] * w1row + b1row)
    ab2 = an * b2[None, :]
    bc = ac * jnp.ones((1, D), f32)
    w2p = jnp.zeros((D, D), f32).at[:16, :].set(w2)
    consts = (jnp.ones((1, NV), f32), m1, m2, ab2, bc, w2p,
              jnp.full((D, 2), 1.0 / D, f32),
              gamma.reshape(1, D), beta.reshape(1, D))
    vidf = variate_ids.astype(f32).reshape(P, 1)
    cidf = cat_ids.astype(f32).reshape(P, 1)
    valf = value_num.reshape(P, 1)

    out = None
    for s in range(SLICES):
        off = s * NBLK_S

        def omap(i, off=off):
            return (off + i, 0)

        aliased = out is not None
        call = pl.pallas_call(
            _tc_body_nop if aliased else _tc_body,
            grid=(NBLK_S,),
            in_specs=_tc_specs(s, aliased),
            out_specs=pl.BlockSpec((TC_BLK, D), omap),
            out_shape=jax.ShapeDtypeStruct((P, D), f32),
            input_output_aliases={13: 0} if aliased else {},
            interpret=interpret,
        )
        args = (vidf, cidf, valf, e_cats[s]) + consts
        if aliased:
            args = args + (out,)
        out = call(*args)
    return out


def kernel(variate_ids, value_num, cat_ids, variate_type, numeric_means,
           numeric_stds, w1, b1, w2, b2, cat_table, gamma, beta):
    gather_ids = jnp.maximum(cat_ids.reshape(P), 0)
    sc = _sc_gather_call()
    e_cats = [sc(lax.slice_in_dim(gather_ids, s * PS, (s + 1) * PS), cat_table)
              for s in range(SLICES)]

    out = _tc_forward(e_cats, variate_ids, value_num, cat_ids, variate_type,
                      numeric_means, numeric_stds, w1, b1, w2, b2, gamma,
                      beta)
    return out.reshape(B, T, D)


# speedup vs baseline: 1.4669x; 1.2317x over previous
import functools

import jax
import jax.numpy as jnp
from jax import lax
from jax.experimental import pallas as pl
from jax.experimental.pallas import tpu as pltpu
from jax.experimental.pallas import tpu_sc as plsc

D = 128
NV = 64
B = 1024
T = 200
P = B * T
NUM_CORES = 2
NUM_SUBCORES = 16
NW = NUM_CORES * NUM_SUBCORES
SLICES = 5
PS = P // SLICES
PW = PS // NW
CHUNK = 128
NSLOT = 6
NPRE = NSLOT - 2
NCHUNK = PW // CHUNK
TC_BLK = 5120
NBLK_S = PS // TC_BLK
NBLK = P // TC_BLK


def _sc_gather(cid_hbm, table_hbm, ecat_hbm, idx_v, rows_v, gsem, wsem):
    wid = lax.axis_index("s") * NUM_CORES + lax.axis_index("c")
    base = wid * PW
    pltpu.sync_copy(cid_hbm.at[pl.ds(base, PW)], idx_v)

    def g_start(j, slot):
        pltpu.async_copy(
            table_hbm.at[idx_v.at[pl.ds(j * CHUNK, CHUNK)]],
            rows_v.at[slot], gsem)

    def g_wait(slot):
        pltpu.make_async_copy(
            table_hbm.at[idx_v.at[pl.ds(0, CHUNK)]],
            rows_v.at[slot], gsem).wait()

    def w_start(j, slot):
        pltpu.async_copy(
            rows_v.at[slot], ecat_hbm.at[pl.ds(base + j * CHUNK, CHUNK)],
            wsem)

    def w_wait(slot):
        pltpu.make_async_copy(
            rows_v.at[slot], ecat_hbm.at[pl.ds(base, CHUNK)], wsem).wait()

    for k in range(min(NPRE, NCHUNK)):
        g_start(k, k % NSLOT)

    def body(j, carry):
        slot = lax.rem(j, NSLOT)

        @pl.when(j >= 2)
        def _():
            w_wait(lax.rem(j - 2, NSLOT))

        @pl.when(j + NPRE < NCHUNK)
        def _():
            g_start(j + NPRE, lax.rem(j + NPRE, NSLOT))

        g_wait(slot)
        w_start(j, slot)
        return carry

    lax.fori_loop(0, NCHUNK, body, 0)
    w_wait(lax.rem(NCHUNK - 2, NSLOT))
    w_wait(lax.rem(NCHUNK - 1, NSLOT))


@functools.lru_cache(maxsize=None)
def _sc_gather_call():
    return pl.kernel(
        _sc_gather,
        out_type=jax.ShapeDtypeStruct((PS, D), jnp.float32),
        mesh=plsc.VectorSubcoreMesh(
            core_axis_name="c", subcore_axis_name="s",
            num_cores=NUM_CORES, num_subcores=NUM_SUBCORES,
        ),
        scratch_types=[
            pltpu.VMEM((PW,), jnp.int32),
            pltpu.VMEM((NSLOT, CHUNK, D), jnp.float32),
            pltpu.SemaphoreType.DMA,
            pltpu.SemaphoreType.DMA,
        ],
    )


def _tc_body(vidf_ref, val_ref, ecat_ref, ones64_ref, ones128_ref,
             cols4_ref, w1r_ref, b1r_ref, w2_ref, b2r_ref, invd_ref, g_ref,
             be_ref, out_ref):
    vid = vidf_ref[:, :]
    val = val_ref[:, :]
    f32 = jnp.float32
    dot = lambda a, b: jnp.dot(a, b, preferred_element_type=f32)
    vidb = dot(vid, ones64_ref[:, :])
    eq = vidb == lax.broadcasted_iota(jnp.int32, (TC_BLK, NV), 1).astype(f32)
    oh = jnp.where(eq, 1.0, 0.0)
    scal = dot(oh, cols4_ref[:, :])
    isg = scal[:, 0:1]
    misg = scal[:, 1:2]
    an = scal[:, 2:3]
    xstd = val * isg - misg
    h = jnp.maximum(dot(xstd, w1r_ref[:, :]) + b1r_ref[0:1, :], 0.0)
    e_num = dot(h, w2_ref[:, :]) + b2r_ref[0:1, :]
    anb = dot(an, ones128_ref[:, :])
    ecat = ecat_ref[:, :]
    e_val = ecat + anb * (e_num - ecat)
    m = dot(e_val, invd_ref[:, 0:1])
    s2 = dot(e_val * e_val, invd_ref[:, 1:2])
    r = lax.rsqrt(s2 - m * m + 1e-5)
    rb = dot(r, g_ref[:, :])
    cb = dot(-m * r, g_ref[:, :])
    out_ref[:, :] = e_val * rb + (cb + be_ref[0:1, :])


def _small2d(shape):
    return pl.BlockSpec(shape, lambda i: (0,) * len(shape))


def _tc_specs(s, aliased):
    off = s * NBLK_S
    specs = [
        pl.BlockSpec((TC_BLK, 1), lambda i: (off + i, 0)),
        pl.BlockSpec((TC_BLK, 1), lambda i: (off + i, 0)),
        pl.BlockSpec((TC_BLK, D), lambda i: (i, 0)),
        _small2d((1, NV)),
        _small2d((1, D)),
        _small2d((NV, 4)),
        _small2d((1, 16)),
        _small2d((1, 16)),
        _small2d((16, D)),
        _small2d((1, D)),
        _small2d((D, 2)),
        _small2d((1, D)),
        _small2d((1, D)),
    ]
    if aliased:
        specs.append(pl.BlockSpec(memory_space=pl.ANY))
    return specs


def _tc_body_nop(*refs):
    _tc_body(*refs[:13], refs[-1])


def _tc_forward(e_cats, variate_ids, value_num, cat_ids, variate_type,
                numeric_means, numeric_stds, w1, b1, w2, b2, gamma, beta,
                interpret=False):
    f32 = jnp.float32
    isg = 1.0 / (numeric_stds + 1e-6)
    misg = numeric_means * isg
    an = (variate_type == 0).astype(f32)
    cols4 = jnp.stack([isg, misg, an, jnp.zeros((NV,), f32)], axis=1)
    consts = (jnp.ones((1, NV), f32), jnp.ones((1, D), f32), cols4,
              w1.reshape(1, 16), b1.reshape(1, 16), w2,
              b2.reshape(1, D), jnp.full((D, 2), 1.0 / D, f32),
              gamma.reshape(1, D), beta.reshape(1, D))
    vidf = variate_ids.astype(f32).reshape(P, 1)
    valf = value_num.reshape(P, 1)

    out = None
    for s in range(SLICES):
        off = s * NBLK_S

        def omap(i, off=off):
            return (off + i, 0)

        aliased = out is not None
        call = pl.pallas_call(
            _tc_body_nop if aliased else _tc_body,
            grid=(NBLK_S,),
            in_specs=_tc_specs(s, aliased),
            out_specs=pl.BlockSpec((TC_BLK, D), omap),
            out_shape=jax.ShapeDtypeStruct((P, D), f32),
            input_output_aliases={13: 0} if aliased else {},
            interpret=interpret,
        )
        args = (vidf, valf, e_cats[s]) + consts
        if aliased:
            args = args + (out,)
        out = call(*args)
    return out


def kernel(variate_ids, value_num, cat_ids, variate_type, numeric_means,
           numeric_stds, w1, b1, w2, b2, cat_table, gamma, beta):
    gather_ids = jnp.maximum(cat_ids.reshape(P), 0)
    sc = _sc_gather_call()
    e_cats = [sc(lax.slice_in_dim(gather_ids, s * PS, (s + 1) * PS), cat_table)
              for s in range(SLICES)]

    out = _tc_forward(e_cats, variate_ids, value_num, cat_ids, variate_type,
                      numeric_means, numeric_stds, w1, b1, w2, b2, gamma,
                      beta)
    return out.reshape(B, T, D)


# speedup vs baseline: 1.4751x; 1.0056x over previous
import functools

import jax
import jax.numpy as jnp
from jax import lax
from jax.experimental import pallas as pl
from jax.experimental.pallas import tpu as pltpu
from jax.experimental.pallas import tpu_sc as plsc

D = 128
NV = 64
B = 1024
T = 200
P = B * T
NUM_CORES = 2
NUM_SUBCORES = 16
NW = NUM_CORES * NUM_SUBCORES
SLICES = 5
PS = P // SLICES
PW = PS // NW
WB = 2
GROUPS = PW // 128
NGRP = GROUPS // WB
NSLOT = 3
TC_BLK = 5120
NBLK_S = PS // TC_BLK
NBLK = P // TC_BLK


def _sc_gather(cid_hbm, table_hbm, ecat_hbm, idx_v, rows_v, gsem, wsem):
    wid = lax.axis_index("s") * NUM_CORES + lax.axis_index("c")
    base = wid * GROUPS
    pltpu.sync_copy(cid_hbm.at[pl.ds(wid * PW, PW)], idx_v)

    def g_start(i, slot):
        for k in range(WB):
            pltpu.async_copy(
                table_hbm.at[idx_v.at[pl.ds((i * WB + k) * 128, 128)]],
                rows_v.at[slot, k], gsem)

    def g_wait(slot):
        for k in range(WB):
            pltpu.make_async_copy(
                table_hbm.at[idx_v.at[pl.ds(0, 128)]],
                rows_v.at[slot, k], gsem).wait()

    def w_start(i, slot):
        pltpu.async_copy(
            rows_v.at[slot], ecat_hbm.at[pl.ds(base + i * WB, WB)],
            wsem)

    def w_wait(slot):
        pltpu.make_async_copy(
            rows_v.at[slot], ecat_hbm.at[pl.ds(base, WB)], wsem).wait()

    g_start(0, 0)

    def body(i, carry):
        slot = lax.rem(i, NSLOT)

        @pl.when(i >= 2)
        def _():
            w_wait(lax.rem(i - 2, NSLOT))

        @pl.when(i + 1 < NGRP)
        def _():
            g_start(i + 1, lax.rem(i + 1, NSLOT))

        g_wait(slot)
        w_start(i, slot)
        return carry

    lax.fori_loop(0, NGRP, body, 0)
    w_wait(lax.rem(NGRP - 2, NSLOT))
    w_wait(lax.rem(NGRP - 1, NSLOT))


@functools.lru_cache(maxsize=None)
def _sc_gather_call():
    return pl.kernel(
        _sc_gather,
        out_type=jax.ShapeDtypeStruct((PS // 128, 128, D), jnp.float32),
        mesh=plsc.VectorSubcoreMesh(
            core_axis_name="c", subcore_axis_name="s",
            num_cores=NUM_CORES, num_subcores=NUM_SUBCORES,
        ),
        scratch_types=[
            pltpu.VMEM((PW,), jnp.int32),
            pltpu.VMEM((NSLOT, WB, 128, D), jnp.float32),
            pltpu.SemaphoreType.DMA,
            pltpu.SemaphoreType.DMA,
        ],
    )


def _tc_body(vidf_ref, val_ref, ecat_ref, ones64_ref, ones128_ref,
             cols4_ref, w1r_ref, b1r_ref, w2_ref, b2r_ref, invd_ref, g_ref,
             be_ref, out_ref):
    vid = vidf_ref[:, :]
    val = val_ref[:, :]
    f32 = jnp.float32
    dot = lambda a, b: jnp.dot(a, b, preferred_element_type=f32)
    vidb = dot(vid, ones64_ref[:, :])
    eq = vidb == lax.broadcasted_iota(jnp.int32, (TC_BLK, NV), 1).astype(f32)
    oh = jnp.where(eq, 1.0, 0.0)
    scal = dot(oh, cols4_ref[:, :])
    isg = scal[:, 0:1]
    misg = scal[:, 1:2]
    an = scal[:, 2:3]
    xstd = val * isg - misg
    h = jnp.maximum(dot(xstd, w1r_ref[:, :]) + b1r_ref[0:1, :], 0.0)
    e_num = dot(h, w2_ref[:, :]) + b2r_ref[0:1, :]
    anb = dot(an, ones128_ref[:, :])
    ecat = ecat_ref[:, :]
    e_val = ecat + anb * (e_num - ecat)
    m = dot(e_val, invd_ref[:, 0:1])
    s2 = dot(e_val * e_val, invd_ref[:, 1:2])
    r = lax.rsqrt(s2 - m * m + 1e-5)
    rb = dot(r, g_ref[:, :])
    cb = dot(-m * r, g_ref[:, :])
    out_ref[:, :] = e_val * rb + (cb + be_ref[0:1, :])


def _small2d(shape):
    return pl.BlockSpec(shape, lambda i: (0,) * len(shape))


def _tc_specs(s, aliased):
    off = s * NBLK_S
    specs = [
        pl.BlockSpec((TC_BLK, 1), lambda i: (off + i, 0)),
        pl.BlockSpec((TC_BLK, 1), lambda i: (off + i, 0)),
        pl.BlockSpec((TC_BLK, D), lambda i: (i, 0)),
        _small2d((1, NV)),
        _small2d((1, D)),
        _small2d((NV, 4)),
        _small2d((1, 16)),
        _small2d((1, 16)),
        _small2d((16, D)),
        _small2d((1, D)),
        _small2d((D, 2)),
        _small2d((1, D)),
        _small2d((1, D)),
    ]
    if aliased:
        specs.append(pl.BlockSpec(memory_space=pl.ANY))
    return specs


def _tc_body_nop(*refs):
    _tc_body(*refs[:13], refs[-1])


def _tc_forward(e_cats, variate_ids, value_num, cat_ids, variate_type,
                numeric_means, numeric_stds, w1, b1, w2, b2, gamma, beta,
                interpret=False):
    f32 = jnp.float32
    isg = 1.0 / (numeric_stds + 1e-6)
    misg = numeric_means * isg
    an = (variate_type == 0).astype(f32)
    cols4 = jnp.stack([isg, misg, an, jnp.zeros((NV,), f32)], axis=1)
    consts = (jnp.ones((1, NV), f32), jnp.ones((1, D), f32), cols4,
              w1.reshape(1, 16), b1.reshape(1, 16), w2,
              b2.reshape(1, D), jnp.full((D, 2), 1.0 / D, f32),
              gamma.reshape(1, D), beta.reshape(1, D))
    vidf = variate_ids.astype(f32).reshape(P, 1)
    valf = value_num.reshape(P, 1)

    out = None
    for s in range(SLICES):
        off = s * NBLK_S

        def omap(i, off=off):
            return (off + i, 0)

        aliased = out is not None
        call = pl.pallas_call(
            _tc_body_nop if aliased else _tc_body,
            grid=(NBLK_S,),
            in_specs=_tc_specs(s, aliased),
            out_specs=pl.BlockSpec((TC_BLK, D), omap),
            out_shape=jax.ShapeDtypeStruct((P, D), f32),
            input_output_aliases={13: 0} if aliased else {},
            interpret=interpret,
        )
        args = (vidf, valf, e_cats[s]) + consts
        if aliased:
            args = args + (out,)
        out = call(*args)
    return out


def kernel(variate_ids, value_num, cat_ids, variate_type, numeric_means,
           numeric_stds, w1, b1, w2, b2, cat_table, gamma, beta):
    gather_ids = jnp.maximum(cat_ids.reshape(P), 0)
    sc = _sc_gather_call()
    e_cats = [sc(lax.slice_in_dim(gather_ids, s * PS, (s + 1) * PS),
                 cat_table).reshape(PS, D)
              for s in range(SLICES)]

    out = _tc_forward(e_cats, variate_ids, value_num, cat_ids, variate_type,
                      numeric_means, numeric_stds, w1, b1, w2, b2, gamma,
                      beta)
    return out.reshape(B, T, D)


# speedup vs baseline: 1.5353x; 1.0408x over previous
import functools

import jax
import jax.numpy as jnp
from jax import lax
from jax.experimental import pallas as pl
from jax.experimental.pallas import tpu as pltpu
from jax.experimental.pallas import tpu_sc as plsc

D = 128
NV = 64
B = 1024
T = 200
P = B * T
NUM_CORES = 2
NUM_SUBCORES = 16
NW = NUM_CORES * NUM_SUBCORES
SLICES = 1
PS = P // SLICES
PW = PS // NW
WB = 2
GROUPS = PW // 128
NGRP = GROUPS // WB
NSLOT = 3
TC_BLK = 5120
NBLK_S = PS // TC_BLK
NBLK = P // TC_BLK


def _sc_gather(cid_hbm, table_hbm, ecat_hbm, idx_v, rows_v, gsem, wsem):
    wid = lax.axis_index("s") * NUM_CORES + lax.axis_index("c")
    base = wid * GROUPS
    pltpu.sync_copy(cid_hbm.at[pl.ds(wid * PW, PW)], idx_v)

    def g_start(i, slot):
        for k in range(WB):
            pltpu.async_copy(
                table_hbm.at[idx_v.at[pl.ds((i * WB + k) * 128, 128)]],
                rows_v.at[slot, k], gsem)

    def g_wait(slot):
        for k in range(WB):
            pltpu.make_async_copy(
                table_hbm.at[idx_v.at[pl.ds(0, 128)]],
                rows_v.at[slot, k], gsem).wait()

    def w_start(i, slot):
        pltpu.async_copy(
            rows_v.at[slot], ecat_hbm.at[pl.ds(base + i * WB, WB)],
            wsem)

    def w_wait(slot):
        pltpu.make_async_copy(
            rows_v.at[slot], ecat_hbm.at[pl.ds(base, WB)], wsem).wait()

    g_start(0, 0)

    def body(i, carry):
        slot = lax.rem(i, NSLOT)

        @pl.when(i >= 2)
        def _():
            w_wait(lax.rem(i - 2, NSLOT))

        @pl.when(i + 1 < NGRP)
        def _():
            g_start(i + 1, lax.rem(i + 1, NSLOT))

        g_wait(slot)
        w_start(i, slot)
        return carry

    lax.fori_loop(0, NGRP, body, 0)
    w_wait(lax.rem(NGRP - 2, NSLOT))
    w_wait(lax.rem(NGRP - 1, NSLOT))


@functools.lru_cache(maxsize=None)
def _sc_gather_call():
    return pl.kernel(
        _sc_gather,
        out_type=jax.ShapeDtypeStruct((PS // 128, 128, D), jnp.float32),
        mesh=plsc.VectorSubcoreMesh(
            core_axis_name="c", subcore_axis_name="s",
            num_cores=NUM_CORES, num_subcores=NUM_SUBCORES,
        ),
        scratch_types=[
            pltpu.VMEM((PW,), jnp.int32),
            pltpu.VMEM((NSLOT, WB, 128, D), jnp.float32),
            pltpu.SemaphoreType.DMA,
            pltpu.SemaphoreType.DMA,
        ],
    )


def _tc_body(vidf_ref, val_ref, ecat_ref, ones64_ref, ones128_ref,
             cols4_ref, w1r_ref, b1r_ref, w2_ref, b2r_ref, invd_ref, g_ref,
             be_ref, out_ref):
    vid = vidf_ref[:, :]
    val = val_ref[:, :]
    f32 = jnp.float32
    dot = lambda a, b: jnp.dot(a, b, preferred_element_type=f32)
    vidb = dot(vid, ones64_ref[:, :])
    eq = vidb == lax.broadcasted_iota(jnp.int32, (TC_BLK, NV), 1).astype(f32)
    oh = jnp.where(eq, 1.0, 0.0)
    scal = dot(oh, cols4_ref[:, :])
    isg = scal[:, 0:1]
    misg = scal[:, 1:2]
    an = scal[:, 2:3]
    xstd = val * isg - misg
    h = jnp.maximum(dot(xstd, w1r_ref[:, :]) + b1r_ref[0:1, :], 0.0)
    e_num = dot(h, w2_ref[:, :]) + b2r_ref[0:1, :]
    anb = dot(an, ones128_ref[:, :])
    ecat = ecat_ref[:, :]
    e_val = ecat + anb * (e_num - ecat)
    m = dot(e_val, invd_ref[:, 0:1])
    s2 = dot(e_val * e_val, invd_ref[:, 1:2])
    r = lax.rsqrt(s2 - m * m + 1e-5)
    rb = dot(r, g_ref[:, :])
    cb = dot(-m * r, g_ref[:, :])
    out_ref[:, :] = e_val * rb + (cb + be_ref[0:1, :])


def _small2d(shape):
    return pl.BlockSpec(shape, lambda i: (0,) * len(shape))


def _tc_specs(s, aliased):
    off = s * NBLK_S
    specs = [
        pl.BlockSpec((TC_BLK, 1), lambda i: (off + i, 0)),
        pl.BlockSpec((TC_BLK, 1), lambda i: (off + i, 0)),
        pl.BlockSpec((TC_BLK, D), lambda i: (i, 0)),
        _small2d((1, NV)),
        _small2d((1, D)),
        _small2d((NV, 4)),
        _small2d((1, 16)),
        _small2d((1, 16)),
        _small2d((16, D)),
        _small2d((1, D)),
        _small2d((D, 2)),
        _small2d((1, D)),
        _small2d((1, D)),
    ]
    if aliased:
        specs.append(pl.BlockSpec(memory_space=pl.ANY))
    return specs


def _tc_body_nop(*refs):
    _tc_body(*refs[:13], refs[-1])


def _tc_forward(e_cats, variate_ids, value_num, cat_ids, variate_type,
                numeric_means, numeric_stds, w1, b1, w2, b2, gamma, beta,
                interpret=False):
    f32 = jnp.float32
    isg = 1.0 / (numeric_stds + 1e-6)
    misg = numeric_means * isg
    an = (variate_type == 0).astype(f32)
    cols4 = jnp.stack([isg, misg, an, jnp.zeros((NV,), f32)], axis=1)
    consts = (jnp.ones((1, NV), f32), jnp.ones((1, D), f32), cols4,
              w1.reshape(1, 16), b1.reshape(1, 16), w2,
              b2.reshape(1, D), jnp.full((D, 2), 1.0 / D, f32),
              gamma.reshape(1, D), beta.reshape(1, D))
    vidf = variate_ids.astype(f32).reshape(P, 1)
    valf = value_num.reshape(P, 1)

    out = None
    for s in range(SLICES):
        off = s * NBLK_S

        def omap(i, off=off):
            return (off + i, 0)

        aliased = out is not None
        call = pl.pallas_call(
            _tc_body_nop if aliased else _tc_body,
            grid=(NBLK_S,),
            in_specs=_tc_specs(s, aliased),
            out_specs=pl.BlockSpec((TC_BLK, D), omap),
            out_shape=jax.ShapeDtypeStruct((P, D), f32),
            input_output_aliases={13: 0} if aliased else {},
            interpret=interpret,
        )
        args = (vidf, valf, e_cats[s]) + consts
        if aliased:
            args = args + (out,)
        out = call(*args)
    return out


def kernel(variate_ids, value_num, cat_ids, variate_type, numeric_means,
           numeric_stds, w1, b1, w2, b2, cat_table, gamma, beta):
    gather_ids = jnp.maximum(cat_ids.reshape(P), 0)
    sc = _sc_gather_call()
    e_cats = [sc(lax.slice_in_dim(gather_ids, s * PS, (s + 1) * PS),
                 cat_table).reshape(PS, D)
              for s in range(SLICES)]

    out = _tc_forward(e_cats, variate_ids, value_num, cat_ids, variate_type,
                      numeric_means, numeric_stds, w1, b1, w2, b2, gamma,
                      beta)
    return out.reshape(B, T, D)
